# Initial kernel scaffold; baseline (speedup 1.0000x reference)
import jax
import jax.numpy as jnp
from jax.experimental import pallas as pl


def kernel(x):
    raise NotImplementedError("write your pallas kernel here")


# speedup vs baseline: 17.7197x; 17.7197x over previous
import jax
import jax.numpy as jnp
from jax.experimental import pallas as pl

_K = 256


def _topk_mask_kernel(x_ref, o_ref):
    x = x_ref[...]
    b = jax.lax.bitcast_convert_type(x, jnp.int32)
    key = b ^ ((b >> 31) & jnp.int32(0x7FFFFFFF))

    cnt_pos = jnp.sum((key >= 0).astype(jnp.int32), axis=1, keepdims=True)
    t = jnp.where(cnt_pos >= _K, jnp.int32(0), jnp.int32(-2147483648))

    def body(i, t):
        bit = 30 - i
        cand = t | (jnp.int32(1) << bit)
        cnt = jnp.sum((key >= cand).astype(jnp.int32), axis=1, keepdims=True)
        return jnp.where(cnt >= _K, cand, t)

    t = jax.lax.fori_loop(0, 31, body, t)

    thr_bits = t ^ ((t >> 31) & jnp.int32(0x7FFFFFFF))
    thr = jax.lax.bitcast_convert_type(thr_bits, jnp.float32)
    o_ref[...] = jnp.where(x >= thr, x, jnp.zeros_like(x))


@jax.jit
def kernel(x):
    return pl.pallas_call(
        _topk_mask_kernel,
        out_shape=jax.ShapeDtypeStruct(x.shape, x.dtype),
    )(x)
